# baseline (speedup 1.0000x reference)
import functools

import jax
import jax.numpy as jnp
from jax import lax
from jax.experimental import pallas as pl
from jax.experimental.pallas import tpu as pltpu
from jax.experimental.pallas import tpu_sc as plsc

N = 10000
E = 160000
D = 256
H = 128
NC = 2
NT = 16
CH = 104
NCHK = 98
NPAIR = NCHK // 2
LT = NCHK * CH
E_PAD = NT * LT
RPT = 632
NPAD = NT * RPT


def _make_sc_aggregate():
    mesh = plsc.VectorSubcoreMesh(core_axis_name="c", subcore_axis_name="s",
                                  num_cores=NC, num_subcores=NT)

    @functools.partial(
        pl.kernel,
        out_type=jax.ShapeDtypeStruct((NC, NPAD, H), jnp.float32),
        mesh=mesh,
        scratch_types=[
            pltpu.VMEM((LT,), jnp.int32),
            pltpu.VMEM((NCHK, CH), jnp.int32),
            pltpu.VMEM((2 * CH, H), jnp.float32),
            pltpu.VMEM_SHARED((NPAD, H), jnp.float32),
            pltpu.SemaphoreType.DMA,
        ],
    )
    def agg_kernel(xflat, srcp, dstp, zrows, out, idx_v, dst_v, rows_v,
                   accum, sem):
        c = lax.axis_index("c")
        s = lax.axis_index("s")
        pltpu.sync_copy(srcp.at[c, s], idx_v)
        pltpu.sync_copy(dstp.at[s], dst_v)
        pltpu.sync_copy(zrows, accum.at[pl.ds(s * RPT, RPT)])
        plsc.subcore_barrier()

        def body(k, carry):
            j2 = 2 * k
            off = pl.multiple_of(j2 * CH, 8)
            pltpu.async_copy(xflat.at[idx_v.at[pl.ds(off, 2 * CH)]], rows_v,
                             sem).wait()
            pltpu.sync_copy(rows_v.at[pl.ds(0, CH)],
                            accum.at[dst_v.at[j2]], add=True)
            pltpu.sync_copy(rows_v.at[pl.ds(CH, CH)],
                            accum.at[dst_v.at[j2 + 1]], add=True)
            return carry

        lax.fori_loop(0, NPAIR, body, 0)
        plsc.subcore_barrier()
        pltpu.sync_copy(accum.at[pl.ds(s * RPT, RPT)],
                        out.at[c, pl.ds(s * RPT, RPT)])

    return agg_kernel


_SC_AGG_CACHE = []


def _sc_aggregate(xflat, srcp, dstp, zrows):
    if not _SC_AGG_CACHE:
        _SC_AGG_CACHE.append(_make_sc_aggregate())
    return _SC_AGG_CACHE[0](xflat, srcp, dstp, zrows)


BN = 1000
NB = N // BN


def _mlp_body(x_ref, a_ref, w1_ref, b1_ref, w2_ref, b2_ref, o_ref):
    h = jnp.concatenate([x_ref[0] + a_ref[0], x_ref[1] + a_ref[1]], axis=1)
    h1 = jnp.dot(h, w1_ref[...], preferred_element_type=jnp.float32) + b1_ref[...]
    h1 = jnp.maximum(h1, 0.0)
    h2 = jnp.dot(h1, w2_ref[...], preferred_element_type=jnp.float32) + b2_ref[...]
    h2 = jnp.maximum(h2, 0.0)
    o_ref[0] = h2[:, :H]
    o_ref[1] = h2[:, H:]


def _tc_mlp(x2, agg2, w1, b1, w2, b2):
    return pl.pallas_call(
        _mlp_body,
        grid=(NB,),
        in_specs=[
            pl.BlockSpec((NC, BN, H), lambda i: (0, i, 0)),
            pl.BlockSpec((NC, BN, H), lambda i: (0, i, 0)),
            pl.BlockSpec((D, D), lambda i: (0, 0)),
            pl.BlockSpec((1, D), lambda i: (0, 0)),
            pl.BlockSpec((D, D), lambda i: (0, 0)),
            pl.BlockSpec((1, D), lambda i: (0, 0)),
        ],
        out_specs=pl.BlockSpec((NC, BN, H), lambda i: (0, i, 0)),
        out_shape=jax.ShapeDtypeStruct((NC, NPAD, H), jnp.float32),
    )(x2, agg2, w1, b1, w2, b2)


def _head_body(x_ref, wa_ref, ba_ref, wb_ref, bb_ref, wc_ref, bc_ref,
               wr_ref, br_ref, wk_ref, bk_ref,
               lo_ref, pr_ref, yh_ref, m_ref, z_ref, acc_ref):
    i = pl.program_id(0)

    @pl.when(i == 0)
    def _():
        m_ref[0] = -jnp.inf
        z_ref[0] = 0.0
        acc_ref[...] = jnp.zeros_like(acc_ref)

    h = jnp.concatenate([x_ref[0], x_ref[1]], axis=1)
    a = jnp.tanh(jnp.dot(h, wa_ref[...], preferred_element_type=jnp.float32) + ba_ref[...])
    b = jax.nn.sigmoid(jnp.dot(h, wb_ref[...], preferred_element_type=jnp.float32) + bb_ref[...])
    att = jnp.dot(a * b, wc_ref[...], preferred_element_type=jnp.float32) + bc_ref[...]
    m_old = m_ref[0]
    m_new = jnp.maximum(m_old, jnp.max(att))
    p = jnp.exp(att - m_new)
    scale = jnp.exp(m_old - m_new)
    m_ref[0] = m_new
    z_ref[0] = z_ref[0] * scale + jnp.sum(p)
    pacc = lax.dot_general(p, h, (((0,), (0,)), ((), ())),
                           preferred_element_type=jnp.float32)
    acc_ref[...] = acc_ref[...] * scale + pacc

    @pl.when(i == NB - 1)
    def _():
        hp = acc_ref[...] / z_ref[0]
        r = jnp.maximum(jnp.dot(hp, wr_ref[...], preferred_element_type=jnp.float32) + br_ref[...], 0.0)
        logits = jnp.dot(r, wk_ref[...], preferred_element_type=jnp.float32) + bk_ref[...]
        lo_ref[...] = logits
        pr_ref[...] = jax.nn.sigmoid(logits)
        mx = jnp.max(logits, axis=1, keepdims=True)
        iot = lax.broadcasted_iota(jnp.int32, (1, 4), 1)
        yh_ref[...] = jnp.min(jnp.where(logits >= mx, iot, 4), axis=1,
                              keepdims=True)


def _tc_head(x2, wa, ba, wb, bb, wc, bc, wr, br, wk, bk):
    return pl.pallas_call(
        _head_body,
        grid=(NB,),
        in_specs=[
            pl.BlockSpec((NC, BN, H), lambda i: (0, i, 0)),
            pl.BlockSpec((D, D), lambda i: (0, 0)),
            pl.BlockSpec((1, D), lambda i: (0, 0)),
            pl.BlockSpec((D, D), lambda i: (0, 0)),
            pl.BlockSpec((1, D), lambda i: (0, 0)),
            pl.BlockSpec((D, 1), lambda i: (0, 0)),
            pl.BlockSpec((1, 1), lambda i: (0, 0)),
            pl.BlockSpec((D, D), lambda i: (0, 0)),
            pl.BlockSpec((1, D), lambda i: (0, 0)),
            pl.BlockSpec((D, 4), lambda i: (0, 0)),
            pl.BlockSpec((1, 4), lambda i: (0, 0)),
        ],
        out_specs=[
            pl.BlockSpec((1, 4), lambda i: (0, 0)),
            pl.BlockSpec((1, 4), lambda i: (0, 0)),
            pl.BlockSpec((1, 1), lambda i: (0, 0)),
        ],
        out_shape=[
            jax.ShapeDtypeStruct((1, 4), jnp.float32),
            jax.ShapeDtypeStruct((1, 4), jnp.float32),
            jax.ShapeDtypeStruct((1, 1), jnp.int32),
        ],
        scratch_shapes=[
            pltpu.SMEM((1,), jnp.float32),
            pltpu.SMEM((1,), jnp.float32),
            pltpu.VMEM((1, D), jnp.float32),
        ],
    )(x2, wa, ba, wb, bb, wc, bc, wr, br, wk, bk)


def kernel(feat, edge_latent, params):
    src = edge_latent[0].astype(jnp.int32)
    dst = edge_latent[1].astype(jnp.int32)
    pad_e = E_PAD - E
    src_p = jnp.concatenate([src, jnp.zeros((pad_e,), jnp.int32)])
    dst_p = jnp.concatenate([dst, jnp.full((pad_e,), N, jnp.int32)])
    srcp = jnp.stack([src_p, src_p + NPAD]).reshape(NC, NT, LT)
    dstp = dst_p.reshape(NT, NCHK, CH)
    zrows = jnp.zeros((RPT, H), jnp.float32)

    x2 = jnp.stack([feat[:, :H], feat[:, H:]])
    x2 = jnp.pad(x2, ((0, 0), (0, NPAD - N), (0, 0)))

    p = params
    layers = [
        (p['c1_W1'], p['c1_b1'], p['c1_W2'], p['c1_b2']),
        (p['c2_W1'], p['c2_b1'], p['c2_W2'], p['c2_b2']),
        (p['c3_W1'], p['c3_b1'], p['c3_W2'], p['c3_b2']),
    ]
    for w1, b1, w2, b2 in layers:
        agg2 = _sc_aggregate(x2.reshape(NC * NPAD, H), srcp, dstp, zrows)
        x2 = _tc_mlp(x2, agg2, w1, b1.reshape(1, D), w2, b2.reshape(1, D))

    logits, y_prob, y_hat = _tc_head(
        x2,
        p['att_Wa'], p['att_ba'].reshape(1, D),
        p['att_Wb'], p['att_bb'].reshape(1, D),
        p['att_Wc'], p['att_bc'].reshape(1, 1),
        p['rho_W'], p['rho_b'].reshape(1, D),
        p['cls_W'], p['cls_b'].reshape(1, 4),
    )
    return (logits, y_prob, y_hat)


# speedup vs baseline: 1.2088x; 1.2088x over previous
import functools

import jax
import jax.numpy as jnp
from jax import lax
from jax.experimental import pallas as pl
from jax.experimental.pallas import tpu as pltpu
from jax.experimental.pallas import tpu_sc as plsc

N = 10000
E = 160000
D = 256
H = 128
NC = 2
NT = 16
CH = 128
NCHK = 79
E_PAD = NT * NCHK * CH
RPT = 632
NPAD = NT * RPT


def _make_sc_aggregate():
    mesh = plsc.VectorSubcoreMesh(core_axis_name="c", subcore_axis_name="s",
                                  num_cores=NC, num_subcores=NT)

    @functools.partial(
        pl.kernel,
        out_type=jax.ShapeDtypeStruct((NC, NPAD, H), jnp.float32),
        mesh=mesh,
        scratch_types=[
            pltpu.VMEM((NCHK, CH), jnp.int32),
            pltpu.VMEM((NCHK, CH), jnp.int32),
            pltpu.VMEM((CH, H), jnp.float32),
            pltpu.VMEM_SHARED((NPAD, H), jnp.float32),
            pltpu.SemaphoreType.DMA,
        ],
    )
    def agg_kernel(xflat, srcp, dstp, zrows, out, idx_v, dst_v, rows_v, accum, sem):
        c = lax.axis_index("c")
        s = lax.axis_index("s")
        pltpu.sync_copy(srcp.at[c, s], idx_v)
        pltpu.sync_copy(dstp.at[s], dst_v)
        pltpu.sync_copy(zrows, accum.at[pl.ds(s * RPT, RPT)])
        plsc.subcore_barrier()

        def body(j, carry):
            pltpu.async_copy(xflat.at[idx_v.at[j]], rows_v, sem).wait()
            pltpu.sync_copy(rows_v, accum.at[dst_v.at[j]], add=True)
            return carry

        lax.fori_loop(0, NCHK, body, 0)
        plsc.subcore_barrier()
        pltpu.sync_copy(accum.at[pl.ds(s * RPT, RPT)],
                        out.at[c, pl.ds(s * RPT, RPT)])

    return agg_kernel


_SC_AGG_CACHE = []


def _sc_aggregate(xflat, srcp, dstp, zrows):
    if not _SC_AGG_CACHE:
        _SC_AGG_CACHE.append(_make_sc_aggregate())
    return _SC_AGG_CACHE[0](xflat, srcp, dstp, zrows)


BN = 1000
NB = N // BN


def _mlp_body(x_ref, a_ref, w1_ref, b1_ref, w2_ref, b2_ref, o_ref):
    h = jnp.concatenate([x_ref[0] + a_ref[0], x_ref[1] + a_ref[1]], axis=1)
    h1 = jnp.dot(h, w1_ref[...], preferred_element_type=jnp.float32) + b1_ref[...]
    h1 = jnp.maximum(h1, 0.0)
    h2 = jnp.dot(h1, w2_ref[...], preferred_element_type=jnp.float32) + b2_ref[...]
    h2 = jnp.maximum(h2, 0.0)
    o_ref[0] = h2[:, :H]
    o_ref[1] = h2[:, H:]


def _tc_mlp(x2, agg2, w1, b1, w2, b2):
    return pl.pallas_call(
        _mlp_body,
        grid=(NB,),
        in_specs=[
            pl.BlockSpec((NC, BN, H), lambda i: (0, i, 0)),
            pl.BlockSpec((NC, BN, H), lambda i: (0, i, 0)),
            pl.BlockSpec((D, D), lambda i: (0, 0)),
            pl.BlockSpec((1, D), lambda i: (0, 0)),
            pl.BlockSpec((D, D), lambda i: (0, 0)),
            pl.BlockSpec((1, D), lambda i: (0, 0)),
        ],
        out_specs=pl.BlockSpec((NC, BN, H), lambda i: (0, i, 0)),
        out_shape=jax.ShapeDtypeStruct((NC, NPAD, H), jnp.float32),
    )(x2, agg2, w1, b1, w2, b2)


def _head_body(x_ref, wa_ref, ba_ref, wb_ref, bb_ref, wc_ref, bc_ref,
               wr_ref, br_ref, wk_ref, bk_ref,
               lo_ref, pr_ref, yh_ref, m_ref, z_ref, acc_ref):
    i = pl.program_id(0)

    @pl.when(i == 0)
    def _():
        m_ref[0] = -jnp.inf
        z_ref[0] = 0.0
        acc_ref[...] = jnp.zeros_like(acc_ref)

    h = jnp.concatenate([x_ref[0], x_ref[1]], axis=1)
    a = jnp.tanh(jnp.dot(h, wa_ref[...], preferred_element_type=jnp.float32) + ba_ref[...])
    b = jax.nn.sigmoid(jnp.dot(h, wb_ref[...], preferred_element_type=jnp.float32) + bb_ref[...])
    att = jnp.dot(a * b, wc_ref[...], preferred_element_type=jnp.float32) + bc_ref[...]
    m_old = m_ref[0]
    m_new = jnp.maximum(m_old, jnp.max(att))
    p = jnp.exp(att - m_new)
    scale = jnp.exp(m_old - m_new)
    m_ref[0] = m_new
    z_ref[0] = z_ref[0] * scale + jnp.sum(p)
    pacc = lax.dot_general(p, h, (((0,), (0,)), ((), ())),
                           preferred_element_type=jnp.float32)
    acc_ref[...] = acc_ref[...] * scale + pacc

    @pl.when(i == NB - 1)
    def _():
        hp = acc_ref[...] / z_ref[0]
        r = jnp.maximum(jnp.dot(hp, wr_ref[...], preferred_element_type=jnp.float32) + br_ref[...], 0.0)
        logits = jnp.dot(r, wk_ref[...], preferred_element_type=jnp.float32) + bk_ref[...]
        lo_ref[...] = logits
        pr_ref[...] = jax.nn.sigmoid(logits)
        mx = jnp.max(logits, axis=1, keepdims=True)
        iot = lax.broadcasted_iota(jnp.int32, (1, 4), 1)
        yh_ref[...] = jnp.min(jnp.where(logits >= mx, iot, 4), axis=1,
                              keepdims=True)


def _tc_head(x2, wa, ba, wb, bb, wc, bc, wr, br, wk, bk):
    return pl.pallas_call(
        _head_body,
        grid=(NB,),
        in_specs=[
            pl.BlockSpec((NC, BN, H), lambda i: (0, i, 0)),
            pl.BlockSpec((D, D), lambda i: (0, 0)),
            pl.BlockSpec((1, D), lambda i: (0, 0)),
            pl.BlockSpec((D, D), lambda i: (0, 0)),
            pl.BlockSpec((1, D), lambda i: (0, 0)),
            pl.BlockSpec((D, 1), lambda i: (0, 0)),
            pl.BlockSpec((1, 1), lambda i: (0, 0)),
            pl.BlockSpec((D, D), lambda i: (0, 0)),
            pl.BlockSpec((1, D), lambda i: (0, 0)),
            pl.BlockSpec((D, 4), lambda i: (0, 0)),
            pl.BlockSpec((1, 4), lambda i: (0, 0)),
        ],
        out_specs=[
            pl.BlockSpec((1, 4), lambda i: (0, 0)),
            pl.BlockSpec((1, 4), lambda i: (0, 0)),
            pl.BlockSpec((1, 1), lambda i: (0, 0)),
        ],
        out_shape=[
            jax.ShapeDtypeStruct((1, 4), jnp.float32),
            jax.ShapeDtypeStruct((1, 4), jnp.float32),
            jax.ShapeDtypeStruct((1, 1), jnp.int32),
        ],
        scratch_shapes=[
            pltpu.SMEM((1,), jnp.float32),
            pltpu.SMEM((1,), jnp.float32),
            pltpu.VMEM((1, D), jnp.float32),
        ],
    )(x2, wa, ba, wb, bb, wc, bc, wr, br, wk, bk)


def kernel(feat, edge_latent, params):
    src = edge_latent[0].astype(jnp.int32)
    dst = edge_latent[1].astype(jnp.int32)
    pad_e = E_PAD - E
    src_p = jnp.concatenate([src, jnp.zeros((pad_e,), jnp.int32)])
    dst_p = jnp.concatenate([dst, jnp.full((pad_e,), N, jnp.int32)])
    srcp = jnp.stack([src_p, src_p + NPAD]).reshape(NC, NT, NCHK, CH)
    dstp = dst_p.reshape(NT, NCHK, CH)
    zrows = jnp.zeros((RPT, H), jnp.float32)

    x2 = jnp.stack([feat[:, :H], feat[:, H:]])
    x2 = jnp.pad(x2, ((0, 0), (0, NPAD - N), (0, 0)))

    p = params
    layers = [
        (p['c1_W1'], p['c1_b1'], p['c1_W2'], p['c1_b2']),
        (p['c2_W1'], p['c2_b1'], p['c2_W2'], p['c2_b2']),
        (p['c3_W1'], p['c3_b1'], p['c3_W2'], p['c3_b2']),
    ]
    for w1, b1, w2, b2 in layers:
        agg2 = _sc_aggregate(x2.reshape(NC * NPAD, H), srcp, dstp, zrows)
        x2 = _tc_mlp(x2, agg2, w1, b1.reshape(1, D), w2, b2.reshape(1, D))

    logits, y_prob, y_hat = _tc_head(
        x2,
        p['att_Wa'], p['att_ba'].reshape(1, D),
        p['att_Wb'], p['att_bb'].reshape(1, D),
        p['att_Wc'], p['att_bc'].reshape(1, 1),
        p['rho_W'], p['rho_b'].reshape(1, D),
        p['cls_W'], p['cls_b'].reshape(1, 4),
    )
    return (logits, y_prob, y_hat)


# speedup vs baseline: 1.2161x; 1.0060x over previous
import functools

import jax
import jax.numpy as jnp
from jax import lax
from jax.experimental import pallas as pl
from jax.experimental.pallas import tpu as pltpu
from jax.experimental.pallas import tpu_sc as plsc

N = 10000
E = 160000
D = 256
H = 128
NC = 2
NT = 16
CH = 128
NCHK = 79
E_PAD = NT * NCHK * CH
RPT = 632
NPAD = NT * RPT


def _make_sc_aggregate():
    mesh = plsc.VectorSubcoreMesh(core_axis_name="c", subcore_axis_name="s",
                                  num_cores=NC, num_subcores=NT)

    @functools.partial(
        pl.kernel,
        out_type=jax.ShapeDtypeStruct((NC, NPAD, H), jnp.float32),
        mesh=mesh,
        scratch_types=[
            pltpu.VMEM((NCHK, CH), jnp.int32),
            pltpu.VMEM((NCHK, CH), jnp.int32),
            pltpu.VMEM((CH, H), jnp.float32),
            pltpu.VMEM_SHARED((NPAD, H), jnp.float32),
            pltpu.SemaphoreType.DMA,
        ],
    )
    def agg_kernel(xflat, srcp, dstp, zrows, out, idx_v, dst_v, rows_v, accum, sem):
        c = lax.axis_index("c")
        s = lax.axis_index("s")
        pltpu.sync_copy(srcp.at[c, s], idx_v)
        pltpu.sync_copy(dstp.at[s], dst_v)
        pltpu.sync_copy(zrows, accum.at[pl.ds(s * RPT, RPT)])
        plsc.subcore_barrier()

        def body(j, carry):
            pltpu.async_copy(xflat.at[idx_v.at[j]], rows_v, sem).wait()
            pltpu.sync_copy(rows_v, accum.at[dst_v.at[j]], add=True)
            return carry

        lax.fori_loop(0, NCHK, body, 0)
        plsc.subcore_barrier()
        pltpu.sync_copy(accum.at[pl.ds(s * RPT, RPT)],
                        out.at[c, pl.ds(s * RPT, RPT)])

    return agg_kernel


_SC_AGG_CACHE = []


def _sc_aggregate(xflat, srcp, dstp, zrows):
    if not _SC_AGG_CACHE:
        _SC_AGG_CACHE.append(_make_sc_aggregate())
    return _SC_AGG_CACHE[0](xflat, srcp, dstp, zrows)


BN = 2000
NB = N // BN


def _mlp_body(x_ref, a_ref, w1_ref, b1_ref, w2_ref, b2_ref, o_ref):
    h = jnp.concatenate([x_ref[0] + a_ref[0], x_ref[1] + a_ref[1]], axis=1)
    h1 = jnp.dot(h, w1_ref[...], preferred_element_type=jnp.float32) + b1_ref[...]
    h1 = jnp.maximum(h1, 0.0)
    h2 = jnp.dot(h1, w2_ref[...], preferred_element_type=jnp.float32) + b2_ref[...]
    h2 = jnp.maximum(h2, 0.0)
    o_ref[0] = h2[:, :H]
    o_ref[1] = h2[:, H:]


def _tc_mlp(x2, agg2, w1, b1, w2, b2):
    return pl.pallas_call(
        _mlp_body,
        grid=(NB,),
        in_specs=[
            pl.BlockSpec((NC, BN, H), lambda i: (0, i, 0)),
            pl.BlockSpec((NC, BN, H), lambda i: (0, i, 0)),
            pl.BlockSpec((D, D), lambda i: (0, 0)),
            pl.BlockSpec((1, D), lambda i: (0, 0)),
            pl.BlockSpec((D, D), lambda i: (0, 0)),
            pl.BlockSpec((1, D), lambda i: (0, 0)),
        ],
        out_specs=pl.BlockSpec((NC, BN, H), lambda i: (0, i, 0)),
        out_shape=jax.ShapeDtypeStruct((NC, NPAD, H), jnp.float32),
    )(x2, agg2, w1, b1, w2, b2)


def _head_body(x_ref, wa_ref, ba_ref, wb_ref, bb_ref, wc_ref, bc_ref,
               wr_ref, br_ref, wk_ref, bk_ref,
               lo_ref, pr_ref, yh_ref, m_ref, z_ref, acc_ref):
    i = pl.program_id(0)

    @pl.when(i == 0)
    def _():
        m_ref[0] = -jnp.inf
        z_ref[0] = 0.0
        acc_ref[...] = jnp.zeros_like(acc_ref)

    h = jnp.concatenate([x_ref[0], x_ref[1]], axis=1)
    a = jnp.tanh(jnp.dot(h, wa_ref[...], preferred_element_type=jnp.float32) + ba_ref[...])
    b = jax.nn.sigmoid(jnp.dot(h, wb_ref[...], preferred_element_type=jnp.float32) + bb_ref[...])
    att = jnp.dot(a * b, wc_ref[...], preferred_element_type=jnp.float32) + bc_ref[...]
    m_old = m_ref[0]
    m_new = jnp.maximum(m_old, jnp.max(att))
    p = jnp.exp(att - m_new)
    scale = jnp.exp(m_old - m_new)
    m_ref[0] = m_new
    z_ref[0] = z_ref[0] * scale + jnp.sum(p)
    pacc = lax.dot_general(p, h, (((0,), (0,)), ((), ())),
                           preferred_element_type=jnp.float32)
    acc_ref[...] = acc_ref[...] * scale + pacc

    @pl.when(i == NB - 1)
    def _():
        hp = acc_ref[...] / z_ref[0]
        r = jnp.maximum(jnp.dot(hp, wr_ref[...], preferred_element_type=jnp.float32) + br_ref[...], 0.0)
        logits = jnp.dot(r, wk_ref[...], preferred_element_type=jnp.float32) + bk_ref[...]
        lo_ref[...] = logits
        pr_ref[...] = jax.nn.sigmoid(logits)
        mx = jnp.max(logits, axis=1, keepdims=True)
        iot = lax.broadcasted_iota(jnp.int32, (1, 4), 1)
        yh_ref[...] = jnp.min(jnp.where(logits >= mx, iot, 4), axis=1,
                              keepdims=True)


def _tc_head(x2, wa, ba, wb, bb, wc, bc, wr, br, wk, bk):
    return pl.pallas_call(
        _head_body,
        grid=(NB,),
        in_specs=[
            pl.BlockSpec((NC, BN, H), lambda i: (0, i, 0)),
            pl.BlockSpec((D, D), lambda i: (0, 0)),
            pl.BlockSpec((1, D), lambda i: (0, 0)),
            pl.BlockSpec((D, D), lambda i: (0, 0)),
            pl.BlockSpec((1, D), lambda i: (0, 0)),
            pl.BlockSpec((D, 1), lambda i: (0, 0)),
            pl.BlockSpec((1, 1), lambda i: (0, 0)),
            pl.BlockSpec((D, D), lambda i: (0, 0)),
            pl.BlockSpec((1, D), lambda i: (0, 0)),
            pl.BlockSpec((D, 4), lambda i: (0, 0)),
            pl.BlockSpec((1, 4), lambda i: (0, 0)),
        ],
        out_specs=[
            pl.BlockSpec((1, 4), lambda i: (0, 0)),
            pl.BlockSpec((1, 4), lambda i: (0, 0)),
            pl.BlockSpec((1, 1), lambda i: (0, 0)),
        ],
        out_shape=[
            jax.ShapeDtypeStruct((1, 4), jnp.float32),
            jax.ShapeDtypeStruct((1, 4), jnp.float32),
            jax.ShapeDtypeStruct((1, 1), jnp.int32),
        ],
        scratch_shapes=[
            pltpu.SMEM((1,), jnp.float32),
            pltpu.SMEM((1,), jnp.float32),
            pltpu.VMEM((1, D), jnp.float32),
        ],
    )(x2, wa, ba, wb, bb, wc, bc, wr, br, wk, bk)


def kernel(feat, edge_latent, params):
    src = edge_latent[0].astype(jnp.int32)
    dst = edge_latent[1].astype(jnp.int32)
    pad_e = E_PAD - E
    src_p = jnp.concatenate([src, jnp.zeros((pad_e,), jnp.int32)])
    dst_p = jnp.concatenate([dst, jnp.full((pad_e,), N, jnp.int32)])
    srcp = jnp.stack([src_p, src_p + NPAD]).reshape(NC, NT, NCHK, CH)
    dstp = dst_p.reshape(NT, NCHK, CH)
    zrows = jnp.zeros((RPT, H), jnp.float32)

    x2 = jnp.stack([feat[:, :H], feat[:, H:]])
    x2 = jnp.pad(x2, ((0, 0), (0, NPAD - N), (0, 0)))

    p = params
    layers = [
        (p['c1_W1'], p['c1_b1'], p['c1_W2'], p['c1_b2']),
        (p['c2_W1'], p['c2_b1'], p['c2_W2'], p['c2_b2']),
        (p['c3_W1'], p['c3_b1'], p['c3_W2'], p['c3_b2']),
    ]
    for w1, b1, w2, b2 in layers:
        agg2 = _sc_aggregate(x2.reshape(NC * NPAD, H), srcp, dstp, zrows)
        x2 = _tc_mlp(x2, agg2, w1, b1.reshape(1, D), w2, b2.reshape(1, D))

    logits, y_prob, y_hat = _tc_head(
        x2,
        p['att_Wa'], p['att_ba'].reshape(1, D),
        p['att_Wb'], p['att_bb'].reshape(1, D),
        p['att_Wc'], p['att_bc'].reshape(1, 1),
        p['rho_W'], p['rho_b'].reshape(1, D),
        p['cls_W'], p['cls_b'].reshape(1, 4),
    )
    return (logits, y_prob, y_hat)
